# baseline (speedup 1.0000x reference)
import functools

import jax
import jax.numpy as jnp
from jax import lax
from jax.experimental import pallas as pl
from jax.experimental.pallas import tpu as pltpu
from jax.experimental.pallas import tpu_sc as plsc

_LANES = 16


def kernel(input_ids, word_table, pos_table):
    B, S = input_ids.shape
    V, E = word_table.shape
    N = B * S
    NC, NS = 2, 16
    NW = NC * NS
    per_w = N // NW
    CH = S
    n_chunks = per_w // CH

    ids_flat = input_ids.reshape(N).astype(jnp.int32)

    mesh = plsc.VectorSubcoreMesh(core_axis_name="c", subcore_axis_name="s")

    @functools.partial(
        pl.kernel,
        mesh=mesh,
        out_type=jax.ShapeDtypeStruct((N, E), jnp.float32),
        compiler_params=pltpu.CompilerParams(use_tc_tiling_on_sc=False),
        scratch_types=[
            pltpu.VMEM((CH,), jnp.int32),
            pltpu.VMEM((CH, E), jnp.float32),
            pltpu.VMEM((S, E), jnp.float32),
            pltpu.SemaphoreType.DMA,
        ],
    )
    def emb_kernel(ids_hbm, word_hbm, pos_hbm, out_hbm, idx_v, rows_v, pos_v, sem):
        wid = lax.axis_index("s") * NC + lax.axis_index("c")
        base = wid * per_w
        pltpu.sync_copy(pos_hbm, pos_v)

        def chunk_body(c, carry):
            off = base + c * CH
            pltpu.sync_copy(ids_hbm.at[pl.ds(off, CH)], idx_v)
            pltpu.async_copy(word_hbm.at[idx_v], rows_v, sem).wait()

            def add_body(i, carry2):
                for j in range(E // _LANES):
                    sl = pl.ds(j * _LANES, _LANES)
                    rows_v[i, sl] = rows_v[i, sl] + pos_v[i, sl]
                return carry2

            lax.fori_loop(0, CH, add_body, 0)
            pltpu.sync_copy(rows_v, out_hbm.at[pl.ds(off, CH)])
            return carry

        lax.fori_loop(0, n_chunks, chunk_body, 0)

    out = emb_kernel(ids_flat, word_table, pos_table)
    return out.reshape(B, S, E)


# speedup vs baseline: 1.2071x; 1.2071x over previous
import functools

import jax
import jax.numpy as jnp
from jax import lax
from jax.experimental import pallas as pl
from jax.experimental.pallas import tpu as pltpu
from jax.experimental.pallas import tpu_sc as plsc

_LANES = 16
_NG = 4
_NW = 2


def kernel(input_ids, word_table, pos_table):
    B, S = input_ids.shape
    V, E = word_table.shape
    N = B * S
    NC, NS = 2, 16
    n_workers = NC * NS
    per_w = N // n_workers
    CH = S
    n_chunks = per_w // CH
    n_steps = n_chunks // _NG

    ids_3d = input_ids.reshape(n_workers, n_chunks, CH).astype(jnp.int32)

    mesh = plsc.VectorSubcoreMesh(core_axis_name="c", subcore_axis_name="s")

    @functools.partial(
        pl.kernel,
        mesh=mesh,
        out_type=jax.ShapeDtypeStruct((N, E), jnp.float32),
        compiler_params=pltpu.CompilerParams(use_tc_tiling_on_sc=False),
        scratch_types=[
            pltpu.VMEM((n_chunks, CH), jnp.int32),
            pltpu.VMEM((_NG, CH, E), jnp.float32),
            pltpu.VMEM((_NW, CH, E), jnp.float32),
            pltpu.VMEM((CH, E), jnp.float32),
            pltpu.SemaphoreType.DMA,
            pltpu.SemaphoreType.DMA,
            pltpu.SemaphoreType.DMA,
            pltpu.SemaphoreType.DMA,
            pltpu.SemaphoreType.DMA,
            pltpu.SemaphoreType.DMA,
        ],
    )
    def emb_kernel(ids_hbm, word_hbm, pos_hbm, out_hbm,
                   idx_slab, gbuf, sbuf, pos_v,
                   sg0, sg1, sg2, sg3, sw0, sw1):
        sem_g = [sg0, sg1, sg2, sg3]
        sem_w = [sw0, sw1]
        wid = lax.axis_index("s") * NC + lax.axis_index("c")
        base = wid * per_w

        pltpu.sync_copy(ids_hbm.at[wid], idx_slab)
        pltpu.sync_copy(pos_hbm, pos_v)

        for b in range(_NG):
            pltpu.async_copy(word_hbm.at[idx_slab.at[b]], gbuf.at[b], sem_g[b])

        def step(s, carry):
            for b in range(_NG):
                c = s * _NG + b
                bs = b % _NW
                off = base + c * CH
                g_r = gbuf.at[b]
                s_r = sbuf.at[bs]

                pltpu.make_async_copy(
                    word_hbm.at[idx_slab.at[c]], g_r, sem_g[b]).wait()

                def wait_write():
                    pltpu.make_async_copy(
                        s_r, out_hbm.at[pl.ds(off, CH)], sem_w[bs]).wait()
                if b < _NW:
                    pl.when(s >= 1)(wait_write)
                else:
                    wait_write()

                def add_body(i, carry2):
                    for j in range(E // _LANES):
                        sl = pl.ds(j * _LANES, _LANES)
                        s_r[i, sl] = g_r[i, sl] + pos_v[i, sl]
                    return carry2
                lax.fori_loop(0, CH, add_body, 0)

                def prefetch():
                    pltpu.async_copy(
                        word_hbm.at[idx_slab.at[c + _NG]], g_r, sem_g[b])
                pl.when(s < n_steps - 1)(prefetch)

                pltpu.async_copy(s_r, out_hbm.at[pl.ds(off, CH)], sem_w[bs])
            return carry

        lax.fori_loop(0, n_steps, step, 0)

        for bs in range(_NW):
            pltpu.make_async_copy(
                sbuf.at[bs], out_hbm.at[pl.ds(base, CH)], sem_w[bs]).wait()

    out = emb_kernel(ids_3d, word_table, pos_table)
    return out.reshape(B, S, E)
